# baseline (speedup 1.0000x reference)
import functools

import jax
import jax.numpy as jnp
from jax import lax
from jax.experimental import pallas as pl
from jax.experimental.pallas import tpu as pltpu
from jax.experimental.pallas import tpu_sc as plsc

NC = 2
NS = 16
CW = 128
NBUF = 2


def _sc_mesh():
    return plsc.VectorSubcoreMesh(
        core_axis_name="c", subcore_axis_name="s", num_cores=NC, num_subcores=NS
    )


def _build_deg(e_pad: int, deg_pad: int):
    et = e_pad // (NC * NS)
    assert et % 16 == 0

    @functools.partial(
        pl.kernel,
        out_type=jax.ShapeDtypeStruct((NC * NS, deg_pad), jnp.float32),
        mesh=_sc_mesh(),
        compiler_params=pltpu.CompilerParams(needs_layout_passes=False),
        scratch_types=[
            pltpu.VMEM((et,), jnp.int32),
            pltpu.VMEM((deg_pad,), jnp.float32),
        ],
    )
    def deg_kernel(dst_hbm, out_hbm, idx_v, deg_v):
        c = lax.axis_index("c")
        s = lax.axis_index("s")
        w = c * NS + s
        pltpu.sync_copy(dst_hbm.at[pl.ds(w * et, et)], idx_v)

        @pl.loop(0, deg_pad // 16)
        def _zero(i):
            deg_v[pl.ds(i * 16, 16)] = jnp.zeros((16,), jnp.float32)

        ones = jnp.ones((16,), jnp.float32)

        @pl.loop(0, et // 16)
        def _acc(i):
            idx = idx_v[pl.ds(i * 16, 16)]
            plsc.addupdate_scatter(deg_v, [idx], ones)

        pltpu.sync_copy(deg_v, out_hbm.at[w])

    return deg_kernel


def _build_agg(n: int, e_pad: int, half: int, nrows: int):
    cht = e_pad // (NS * CW)
    rpt = (n // NS) // 8 * 8
    tail = n - rpt * NS
    assert tail % 8 == 0

    @functools.partial(
        pl.kernel,
        out_type=jax.ShapeDtypeStruct((NC, n, half), jnp.float32),
        mesh=_sc_mesh(),
        compiler_params=pltpu.CompilerParams(needs_layout_passes=False),
        scratch_types=[
            pltpu.VMEM((cht, CW), jnp.int32),
            pltpu.VMEM((cht, CW), jnp.int32),
            pltpu.VMEM((CW, half), jnp.float32),
            pltpu.VMEM_SHARED((nrows, half), jnp.float32),
            pltpu.SemaphoreType.DMA,
        ],
    )
    def agg_kernel(hn_hbm, src_hbm, dst_hbm, out_hbm, sidx_v, didx_v, rows_v,
                   acc_sh, sem):
        c = lax.axis_index("c")
        s = lax.axis_index("s")
        tab = hn_hbm.at[c]

        pltpu.sync_copy(src_hbm.at[s], sidx_v)
        pltpu.sync_copy(dst_hbm.at[s], didx_v)
        pltpu.sync_copy(tab.at[pl.ds(s * rpt, rpt)],
                        acc_sh.at[pl.ds(s * rpt, rpt)])
        if tail:
            @pl.when(s == 0)
            def _init_tail():
                pltpu.sync_copy(tab.at[pl.ds(NS * rpt, tail)],
                                acc_sh.at[pl.ds(NS * rpt, tail)])
        plsc.subcore_barrier()

        @pl.loop(0, cht)
        def _chunk(j):
            pltpu.async_copy(tab.at[sidx_v.at[j]], rows_v, sem).wait()
            pltpu.sync_copy(rows_v, acc_sh.at[didx_v.at[j]], add=True)

        plsc.subcore_barrier()
        pltpu.sync_copy(acc_sh.at[pl.ds(s * rpt, rpt)],
                        out_hbm.at[c].at[pl.ds(s * rpt, rpt)])
        if tail:
            @pl.when(s == 0)
            def _out_tail():
                pltpu.sync_copy(acc_sh.at[pl.ds(NS * rpt, tail)],
                                out_hbm.at[c].at[pl.ds(NS * rpt, tail)])

    return agg_kernel


def _dinv_of(deg_blk):
    return lax.rsqrt(jnp.sum(deg_blk, axis=1, keepdims=True) + 1.0)


def _lin1_body(deg_ref, x_ref, w_ref, out_ref):
    dinv = _dinv_of(deg_ref[...])
    h = jnp.dot(x_ref[...], w_ref[...], preferred_element_type=jnp.float32)
    hn = h * dinv
    half = out_ref.shape[2]
    out_ref[0] = hn[:, :half]
    out_ref[1] = hn[:, half:]


def _lin2_body(deg_ref, agg_ref, b_ref, w_ref, out_ref):
    dinv = _dinv_of(deg_ref[...])
    agg = jnp.concatenate([agg_ref[0], agg_ref[1]], axis=1)
    x2 = jnp.maximum(agg * dinv + b_ref[...], 0.0)
    h = jnp.dot(x2, w_ref[...], preferred_element_type=jnp.float32)
    hn = h * dinv
    half = out_ref.shape[2]
    out_ref[0] = hn[:, :half]
    out_ref[1] = hn[:, half:]


def _lin3_body(deg_ref, agg_ref, b_ref, w_ref, bo_ref, out_ref):
    dinv = _dinv_of(deg_ref[...])
    agg = jnp.concatenate([agg_ref[0], agg_ref[1]], axis=1)
    x2 = jnp.maximum(agg * dinv + b_ref[...], 0.0)
    out_ref[...] = (
        jnp.dot(x2, w_ref[...], preferred_element_type=jnp.float32) + bo_ref[...]
    )


def kernel(x, edge_index, W1, b1, W2, b2, Wout, bout):
    n, d_in = x.shape
    d_h = W1.shape[1]
    d_out = Wout.shape[1]
    half = d_h // 2
    e = edge_index.shape[1]

    grp = NS * CW * NBUF
    e_pad = ((e + grp - 1) // grp) * grp
    cht = e_pad // (NS * CW)
    deg_pad = ((n + 1 + 511) // 512) * 512
    nrows = ((n + 1 + 15) // 16) * 16

    src = edge_index[0].astype(jnp.int32)
    dst = edge_index[1].astype(jnp.int32)
    padlen = e_pad - e
    src_p = jnp.concatenate([src, jnp.zeros((padlen,), jnp.int32)])
    dst_p = jnp.concatenate([dst, jnp.full((padlen,), n, jnp.int32)])
    src3 = src_p.reshape(NS, cht, CW)
    dst3 = dst_p.reshape(NS, cht, CW)

    deg_all = _build_deg(e_pad, deg_pad)(dst_p)
    deg_t = deg_all[:, :n].T

    r = 1000
    nblk = n // r
    deg_spec = pl.BlockSpec((r, NC * NS), lambda i: (i, 0))
    halves_spec = pl.BlockSpec((NC, r, half), lambda i: (0, i, 0))
    full_w = lambda shape: pl.BlockSpec(shape, lambda i: (0, 0))

    lin1 = pl.pallas_call(
        _lin1_body,
        grid=(nblk,),
        in_specs=[deg_spec, pl.BlockSpec((r, d_in), lambda i: (i, 0)),
                  full_w((d_in, d_h))],
        out_specs=halves_spec,
        out_shape=jax.ShapeDtypeStruct((NC, n, half), jnp.float32),
    )
    lin2 = pl.pallas_call(
        _lin2_body,
        grid=(nblk,),
        in_specs=[deg_spec, halves_spec, full_w((1, d_h)), full_w((d_h, d_h))],
        out_specs=halves_spec,
        out_shape=jax.ShapeDtypeStruct((NC, n, half), jnp.float32),
    )
    lin3 = pl.pallas_call(
        _lin3_body,
        grid=(nblk,),
        in_specs=[deg_spec, halves_spec, full_w((1, d_h)), full_w((d_h, d_out)),
                  full_w((1, d_out))],
        out_specs=pl.BlockSpec((r, d_out), lambda i: (i, 0)),
        out_shape=jax.ShapeDtypeStruct((n, d_out), jnp.float32),
    )
    agg = _build_agg(n, e_pad, half, nrows)

    hn1 = lin1(deg_t, x, W1)
    agg1 = agg(hn1, src3, dst3)
    hn2 = lin2(deg_t, agg1, b1.reshape(1, -1), W2)
    agg2 = agg(hn2, src3, dst3)
    out = lin3(deg_t, agg2, b2.reshape(1, -1), Wout, bout.reshape(1, -1))
    return out


# speedup vs baseline: 1.7863x; 1.7863x over previous
import functools

import jax
import jax.numpy as jnp
from jax import lax
from jax.experimental import pallas as pl
from jax.experimental.pallas import tpu as pltpu
from jax.experimental.pallas import tpu_sc as plsc

NC = 2
NS = 16
CW = 128
NBUF = 2


def _sc_mesh():
    return plsc.VectorSubcoreMesh(
        core_axis_name="c", subcore_axis_name="s", num_cores=NC, num_subcores=NS
    )


def _build_deg(e_pad: int, deg_pad: int):
    et = e_pad // (NC * NS)
    assert et % 16 == 0

    @functools.partial(
        pl.kernel,
        out_type=jax.ShapeDtypeStruct((NC * NS, deg_pad), jnp.float32),
        mesh=_sc_mesh(),
        compiler_params=pltpu.CompilerParams(needs_layout_passes=False),
        scratch_types=[
            pltpu.VMEM((et,), jnp.int32),
            pltpu.VMEM((deg_pad,), jnp.float32),
        ],
    )
    def deg_kernel(dst_hbm, out_hbm, idx_v, deg_v):
        c = lax.axis_index("c")
        s = lax.axis_index("s")
        w = c * NS + s
        pltpu.sync_copy(dst_hbm.at[pl.ds(w * et, et)], idx_v)

        @pl.loop(0, deg_pad // 16)
        def _zero(i):
            deg_v[pl.ds(i * 16, 16)] = jnp.zeros((16,), jnp.float32)

        ones = jnp.ones((16,), jnp.float32)

        @pl.loop(0, et // 16)
        def _acc(i):
            idx = idx_v[pl.ds(i * 16, 16)]
            plsc.addupdate_scatter(deg_v, [idx], ones)

        pltpu.sync_copy(deg_v, out_hbm.at[w])

    return deg_kernel


def _build_agg(n: int, e_pad: int, half: int, nrows: int):
    cht = e_pad // (NS * CW)
    rpt = (n // NS) // 8 * 8
    tail = n - rpt * NS
    assert tail % 8 == 0

    @functools.partial(
        pl.kernel,
        out_type=jax.ShapeDtypeStruct((NC, n, half), jnp.float32),
        mesh=_sc_mesh(),
        compiler_params=pltpu.CompilerParams(needs_layout_passes=False),
        scratch_types=[
            pltpu.VMEM((cht, CW), jnp.int32),
            pltpu.VMEM((cht, CW), jnp.int32),
            pltpu.VMEM((CW, half), jnp.float32),
            pltpu.VMEM_SHARED((nrows, half), jnp.float32),
            pltpu.SemaphoreType.DMA,
        ],
    )
    def agg_kernel(hn_hbm, src_hbm, dst_hbm, out_hbm, sidx_v, didx_v, rows_v,
                   acc_sh, sem):
        c = lax.axis_index("c")
        s = lax.axis_index("s")
        tab = hn_hbm.at[c]

        pltpu.sync_copy(src_hbm.at[s], sidx_v)
        pltpu.sync_copy(dst_hbm.at[s], didx_v)
        pltpu.sync_copy(tab.at[pl.ds(s * rpt, rpt)],
                        acc_sh.at[pl.ds(s * rpt, rpt)])
        if tail:
            @pl.when(s == 0)
            def _init_tail():
                pltpu.sync_copy(tab.at[pl.ds(NS * rpt, tail)],
                                acc_sh.at[pl.ds(NS * rpt, tail)])
        plsc.subcore_barrier()

        @pl.loop(0, cht)
        def _chunk(j):
            pltpu.async_copy(tab.at[sidx_v.at[j]], rows_v, sem).wait()
            pltpu.sync_copy(rows_v, acc_sh.at[didx_v.at[j]], add=True)

        plsc.subcore_barrier()
        pltpu.sync_copy(acc_sh.at[pl.ds(s * rpt, rpt)],
                        out_hbm.at[c].at[pl.ds(s * rpt, rpt)])
        if tail:
            @pl.when(s == 0)
            def _out_tail():
                pltpu.sync_copy(acc_sh.at[pl.ds(NS * rpt, tail)],
                                out_hbm.at[c].at[pl.ds(NS * rpt, tail)])

    return agg_kernel


def _dinv_of(deg_blk):
    return lax.rsqrt(jnp.sum(deg_blk, axis=1, keepdims=True) + 1.0)


def _lin1_body(deg_ref, x_ref, w_ref, out_ref):
    dinv = _dinv_of(deg_ref[...])
    h = jnp.dot(x_ref[...], w_ref[...], preferred_element_type=jnp.float32)
    hn = h * dinv
    half = out_ref.shape[2]
    out_ref[0] = hn[:, :half]
    out_ref[1] = hn[:, half:]


def _lin2_body(deg_ref, agg_ref, b_ref, w_ref, out_ref):
    dinv = _dinv_of(deg_ref[...])
    agg = jnp.concatenate([agg_ref[0], agg_ref[1]], axis=1)
    x2 = jnp.maximum(agg * dinv + b_ref[...], 0.0)
    h = jnp.dot(x2, w_ref[...], preferred_element_type=jnp.float32)
    hn = h * dinv
    half = out_ref.shape[2]
    out_ref[0] = hn[:, :half]
    out_ref[1] = hn[:, half:]


def _lin3_body(deg_ref, agg_ref, b_ref, w_ref, bo_ref, out_ref):
    dinv = _dinv_of(deg_ref[...])
    agg = jnp.concatenate([agg_ref[0], agg_ref[1]], axis=1)
    x2 = jnp.maximum(agg * dinv + b_ref[...], 0.0)
    out_ref[...] = (
        jnp.dot(x2, w_ref[...], preferred_element_type=jnp.float32) + bo_ref[...]
    )


def kernel(x, edge_index, W1, b1, W2, b2, Wout, bout):
    n, d_in = x.shape
    d_h = W1.shape[1]
    d_out = Wout.shape[1]
    half = d_h // 2
    e = edge_index.shape[1]

    grp = NS * CW
    e_pad = ((e + grp - 1) // grp) * grp
    cht = e_pad // (NS * CW)
    deg_pad = ((n + 1 + 511) // 512) * 512
    nrows = ((n + 1 + 15) // 16) * 16

    src = edge_index[0].astype(jnp.int32)
    dst = edge_index[1].astype(jnp.int32)
    padlen = e_pad - e
    pad_ids = jnp.arange(padlen, dtype=jnp.int32)
    src_p = jnp.concatenate([src, pad_ids % n])
    dst_p = jnp.concatenate([dst, n + pad_ids % (nrows - n)])
    src3 = src_p.reshape(NS, cht, CW)
    dst3 = dst_p.reshape(NS, cht, CW)

    deg_all = _build_deg(e_pad, deg_pad)(dst_p)
    deg_t = deg_all[:, :n].T

    r = 1000
    nblk = n // r
    deg_spec = pl.BlockSpec((r, NC * NS), lambda i: (i, 0))
    halves_spec = pl.BlockSpec((NC, r, half), lambda i: (0, i, 0))
    full_w = lambda shape: pl.BlockSpec(shape, lambda i: (0, 0))

    lin1 = pl.pallas_call(
        _lin1_body,
        grid=(nblk,),
        in_specs=[deg_spec, pl.BlockSpec((r, d_in), lambda i: (i, 0)),
                  full_w((d_in, d_h))],
        out_specs=halves_spec,
        out_shape=jax.ShapeDtypeStruct((NC, n, half), jnp.float32),
    )
    lin2 = pl.pallas_call(
        _lin2_body,
        grid=(nblk,),
        in_specs=[deg_spec, halves_spec, full_w((1, d_h)), full_w((d_h, d_h))],
        out_specs=halves_spec,
        out_shape=jax.ShapeDtypeStruct((NC, n, half), jnp.float32),
    )
    lin3 = pl.pallas_call(
        _lin3_body,
        grid=(nblk,),
        in_specs=[deg_spec, halves_spec, full_w((1, d_h)), full_w((d_h, d_out)),
                  full_w((1, d_out))],
        out_specs=pl.BlockSpec((r, d_out), lambda i: (i, 0)),
        out_shape=jax.ShapeDtypeStruct((n, d_out), jnp.float32),
    )
    agg = _build_agg(n, e_pad, half, nrows)

    hn1 = lin1(deg_t, x, W1)
    agg1 = agg(hn1, src3, dst3)
    hn2 = lin2(deg_t, agg1, b1.reshape(1, -1), W2)
    agg2 = agg(hn2, src3, dst3)
    out = lin3(deg_t, agg2, b2.reshape(1, -1), Wout, bout.reshape(1, -1))
    return out


# speedup vs baseline: 2.6245x; 1.4693x over previous
import functools

import jax
import jax.numpy as jnp
from jax import lax
from jax.experimental import pallas as pl
from jax.experimental.pallas import tpu as pltpu
from jax.experimental.pallas import tpu_sc as plsc

NC = 2
NS = 16
CW = 128
NBUF = 2


def _sc_mesh():
    return plsc.VectorSubcoreMesh(
        core_axis_name="c", subcore_axis_name="s", num_cores=NC, num_subcores=NS
    )


def _build_deg(e_pad: int, deg_pad: int):
    et = e_pad // (NC * NS)
    assert et % 16 == 0

    @functools.partial(
        pl.kernel,
        out_type=jax.ShapeDtypeStruct((NC * NS, deg_pad), jnp.float32),
        mesh=_sc_mesh(),
        compiler_params=pltpu.CompilerParams(needs_layout_passes=False),
        scratch_types=[
            pltpu.VMEM((et,), jnp.int32),
            pltpu.VMEM((deg_pad,), jnp.float32),
        ],
    )
    def deg_kernel(dst_hbm, out_hbm, idx_v, deg_v):
        c = lax.axis_index("c")
        s = lax.axis_index("s")
        w = c * NS + s
        pltpu.sync_copy(dst_hbm.at[pl.ds(w * et, et)], idx_v)

        @pl.loop(0, deg_pad // 16)
        def _zero(i):
            deg_v[pl.ds(i * 16, 16)] = jnp.zeros((16,), jnp.float32)

        ones = jnp.ones((16,), jnp.float32)

        @pl.loop(0, et // 16)
        def _acc(i):
            idx = idx_v[pl.ds(i * 16, 16)]
            plsc.addupdate_scatter(deg_v, [idx], ones)

        pltpu.sync_copy(deg_v, out_hbm.at[w])

    return deg_kernel


def _build_agg(n: int, e_pad: int, half: int, nrows: int):
    cht = e_pad // (NS * CW)
    rpt = (n // NS) // 8 * 8
    tail = n - rpt * NS
    assert tail % 8 == 0

    @functools.partial(
        pl.kernel,
        out_type=jax.ShapeDtypeStruct((NC, n, half), jnp.float32),
        mesh=_sc_mesh(),
        compiler_params=pltpu.CompilerParams(needs_layout_passes=False),
        scratch_types=[
            pltpu.VMEM((cht, CW), jnp.int32),
            pltpu.VMEM((NBUF, CW), jnp.int32),
            pltpu.VMEM((NBUF, CW, half), jnp.float32),
            pltpu.VMEM_SHARED((nrows, half), jnp.float32),
        ] + [pltpu.SemaphoreType.DMA] * (2 * NBUF),
    )
    def agg_kernel(hn_hbm, src_hbm, dst_hbm, out_hbm, sidx_v, didx_v, rows_v,
                   acc_sh, *sems):
        gsems, dsems = sems[:NBUF], sems[NBUF:]
        c = lax.axis_index("c")
        s = lax.axis_index("s")
        tab = hn_hbm.at[c]

        pltpu.sync_copy(src_hbm.at[s], sidx_v)
        pltpu.sync_copy(tab.at[pl.ds(s * rpt, rpt)],
                        acc_sh.at[pl.ds(s * rpt, rpt)])
        if tail:
            @pl.when(s == 0)
            def _init_tail():
                pltpu.sync_copy(tab.at[pl.ds(NS * rpt, tail)],
                                acc_sh.at[pl.ds(NS * rpt, tail)])
        plsc.subcore_barrier()

        pltpu.async_copy(tab.at[sidx_v.at[0]], rows_v.at[0], gsems[0])
        pltpu.async_copy(dst_hbm.at[s].at[0], didx_v.at[0], dsems[0])

        @pl.loop(0, cht, step=NBUF)
        def _chunk(j0):
            for b in range(NBUF):
                j = j0 + b
                nxt = j + 1
                bn = (b + 1) % NBUF

                @pl.when(nxt < cht)
                def _issue():
                    pltpu.async_copy(tab.at[sidx_v.at[nxt]], rows_v.at[bn],
                                     gsems[bn])
                    pltpu.async_copy(dst_hbm.at[s].at[nxt], didx_v.at[bn],
                                     dsems[bn])

                pltpu.make_async_copy(tab.at[sidx_v.at[j]], rows_v.at[b],
                                      gsems[b]).wait()
                pltpu.make_async_copy(dst_hbm.at[s].at[j], didx_v.at[b],
                                      dsems[b]).wait()
                pltpu.sync_copy(rows_v.at[b], acc_sh.at[didx_v.at[b]], add=True)

        plsc.subcore_barrier()
        pltpu.sync_copy(acc_sh.at[pl.ds(s * rpt, rpt)],
                        out_hbm.at[c].at[pl.ds(s * rpt, rpt)])
        if tail:
            @pl.when(s == 0)
            def _out_tail():
                pltpu.sync_copy(acc_sh.at[pl.ds(NS * rpt, tail)],
                                out_hbm.at[c].at[pl.ds(NS * rpt, tail)])

    return agg_kernel


def _dinv_of(deg_blk):
    return lax.rsqrt(jnp.sum(deg_blk, axis=1, keepdims=True) + 1.0)


def _lin1_body(deg_ref, x_ref, w_ref, out_ref):
    dinv = _dinv_of(deg_ref[...])
    h = jnp.dot(x_ref[...], w_ref[...], preferred_element_type=jnp.float32)
    hn = h * dinv
    half = out_ref.shape[2]
    out_ref[0] = hn[:, :half]
    out_ref[1] = hn[:, half:]


def _lin2_body(deg_ref, agg_ref, b_ref, w_ref, out_ref):
    dinv = _dinv_of(deg_ref[...])
    agg = jnp.concatenate([agg_ref[0], agg_ref[1]], axis=1)
    x2 = jnp.maximum(agg * dinv + b_ref[...], 0.0)
    h = jnp.dot(x2, w_ref[...], preferred_element_type=jnp.float32)
    hn = h * dinv
    half = out_ref.shape[2]
    out_ref[0] = hn[:, :half]
    out_ref[1] = hn[:, half:]


def _lin3_body(deg_ref, agg_ref, b_ref, w_ref, bo_ref, out_ref):
    dinv = _dinv_of(deg_ref[...])
    agg = jnp.concatenate([agg_ref[0], agg_ref[1]], axis=1)
    x2 = jnp.maximum(agg * dinv + b_ref[...], 0.0)
    out_ref[...] = (
        jnp.dot(x2, w_ref[...], preferred_element_type=jnp.float32) + bo_ref[...]
    )


def kernel(x, edge_index, W1, b1, W2, b2, Wout, bout):
    n, d_in = x.shape
    d_h = W1.shape[1]
    d_out = Wout.shape[1]
    half = d_h // 2
    e = edge_index.shape[1]

    grp = NS * CW * NBUF
    e_pad = ((e + grp - 1) // grp) * grp
    cht = e_pad // (NS * CW)
    deg_pad = ((n + 1 + 511) // 512) * 512
    nrows = ((n + 1 + 15) // 16) * 16

    src = edge_index[0].astype(jnp.int32)
    dst = edge_index[1].astype(jnp.int32)
    padlen = e_pad - e
    pad_ids = jnp.arange(padlen, dtype=jnp.int32)
    src_p = jnp.concatenate([src, pad_ids % n])
    dst_p = jnp.concatenate([dst, n + pad_ids % (nrows - n)])
    src3 = src_p.reshape(NS, cht, CW)
    dst3 = dst_p.reshape(NS, cht, CW)

    deg_all = _build_deg(e_pad, deg_pad)(dst_p)
    deg_t = deg_all[:, :n].T

    r = 1000
    nblk = n // r
    deg_spec = pl.BlockSpec((r, NC * NS), lambda i: (i, 0))
    halves_spec = pl.BlockSpec((NC, r, half), lambda i: (0, i, 0))
    full_w = lambda shape: pl.BlockSpec(shape, lambda i: (0, 0))

    lin1 = pl.pallas_call(
        _lin1_body,
        grid=(nblk,),
        in_specs=[deg_spec, pl.BlockSpec((r, d_in), lambda i: (i, 0)),
                  full_w((d_in, d_h))],
        out_specs=halves_spec,
        out_shape=jax.ShapeDtypeStruct((NC, n, half), jnp.float32),
    )
    lin2 = pl.pallas_call(
        _lin2_body,
        grid=(nblk,),
        in_specs=[deg_spec, halves_spec, full_w((1, d_h)), full_w((d_h, d_h))],
        out_specs=halves_spec,
        out_shape=jax.ShapeDtypeStruct((NC, n, half), jnp.float32),
    )
    lin3 = pl.pallas_call(
        _lin3_body,
        grid=(nblk,),
        in_specs=[deg_spec, halves_spec, full_w((1, d_h)), full_w((d_h, d_out)),
                  full_w((1, d_out))],
        out_specs=pl.BlockSpec((r, d_out), lambda i: (i, 0)),
        out_shape=jax.ShapeDtypeStruct((n, d_out), jnp.float32),
    )
    agg = _build_agg(n, e_pad, half, nrows)

    hn1 = lin1(deg_t, x, W1)
    agg1 = agg(hn1, src3, dst3)
    hn2 = lin2(deg_t, agg1, b1.reshape(1, -1), W2)
    agg2 = agg(hn2, src3, dst3)
    out = lin3(deg_t, agg2, b2.reshape(1, -1), Wout, bout.reshape(1, -1))
    return out


# speedup vs baseline: 2.6255x; 1.0004x over previous
import functools

import jax
import jax.numpy as jnp
from jax import lax
from jax.experimental import pallas as pl
from jax.experimental.pallas import tpu as pltpu
from jax.experimental.pallas import tpu_sc as plsc

NC = 2
NS = 16
CW = 128
NBUF = 2
NSUB = 2
SW = CW // NSUB


def _sc_mesh():
    return plsc.VectorSubcoreMesh(
        core_axis_name="c", subcore_axis_name="s", num_cores=NC, num_subcores=NS
    )


def _build_deg(e_pad: int, deg_pad: int):
    et = e_pad // (NC * NS)
    assert et % 16 == 0

    @functools.partial(
        pl.kernel,
        out_type=jax.ShapeDtypeStruct((NC * NS, deg_pad), jnp.float32),
        mesh=_sc_mesh(),
        compiler_params=pltpu.CompilerParams(needs_layout_passes=False),
        scratch_types=[
            pltpu.VMEM((et,), jnp.int32),
            pltpu.VMEM((deg_pad,), jnp.float32),
        ],
    )
    def deg_kernel(dst_hbm, out_hbm, idx_v, deg_v):
        c = lax.axis_index("c")
        s = lax.axis_index("s")
        w = c * NS + s
        pltpu.sync_copy(dst_hbm.at[pl.ds(w * et, et)], idx_v)

        @pl.loop(0, deg_pad // 16)
        def _zero(i):
            deg_v[pl.ds(i * 16, 16)] = jnp.zeros((16,), jnp.float32)

        ones = jnp.ones((16,), jnp.float32)

        @pl.loop(0, et // 16)
        def _acc(i):
            idx = idx_v[pl.ds(i * 16, 16)]
            plsc.addupdate_scatter(deg_v, [idx], ones)

        pltpu.sync_copy(deg_v, out_hbm.at[w])

    return deg_kernel


def _build_agg(n: int, e_pad: int, half: int, nrows: int):
    cht = e_pad // (NS * CW)
    rpt = (n // NS) // 8 * 8
    tail = n - rpt * NS
    assert tail % 8 == 0

    @functools.partial(
        pl.kernel,
        out_type=jax.ShapeDtypeStruct((NC, n, half), jnp.float32),
        mesh=_sc_mesh(),
        compiler_params=pltpu.CompilerParams(needs_layout_passes=False),
        scratch_types=[
            pltpu.VMEM((cht, CW), jnp.int32),
            pltpu.VMEM((NBUF, CW), jnp.int32),
            pltpu.VMEM((NBUF, CW, half), jnp.float32),
            pltpu.VMEM_SHARED((nrows, half), jnp.float32),
        ] + [pltpu.SemaphoreType.DMA] * (2 * NBUF),
    )
    def agg_kernel(hn_hbm, src_hbm, dst_hbm, out_hbm, sidx_v, didx_v, rows_v,
                   acc_sh, *sems):
        gsems, dsems = sems[:NBUF], sems[NBUF:]
        c = lax.axis_index("c")
        s = lax.axis_index("s")
        tab = hn_hbm.at[c]

        pltpu.sync_copy(src_hbm.at[s], sidx_v)
        pltpu.sync_copy(tab.at[pl.ds(s * rpt, rpt)],
                        acc_sh.at[pl.ds(s * rpt, rpt)])
        if tail:
            @pl.when(s == 0)
            def _init_tail():
                pltpu.sync_copy(tab.at[pl.ds(NS * rpt, tail)],
                                acc_sh.at[pl.ds(NS * rpt, tail)])
        plsc.subcore_barrier()

        for u in range(NSUB):
            pltpu.async_copy(tab.at[sidx_v.at[0, pl.ds(u * SW, SW)]],
                             rows_v.at[0, pl.ds(u * SW, SW)], gsems[0])
        pltpu.async_copy(dst_hbm.at[s].at[0], didx_v.at[0], dsems[0])

        @pl.loop(0, cht, step=NBUF)
        def _chunk(j0):
            for b in range(NBUF):
                j = j0 + b
                nxt = j + 1
                bn = (b + 1) % NBUF

                @pl.when(nxt < cht)
                def _issue():
                    for u in range(NSUB):
                        pltpu.async_copy(
                            tab.at[sidx_v.at[nxt, pl.ds(u * SW, SW)]],
                            rows_v.at[bn, pl.ds(u * SW, SW)], gsems[bn])
                    pltpu.async_copy(dst_hbm.at[s].at[nxt], didx_v.at[bn],
                                     dsems[bn])

                for u in range(NSUB):
                    pltpu.make_async_copy(
                        tab.at[sidx_v.at[j, pl.ds(u * SW, SW)]],
                        rows_v.at[b, pl.ds(u * SW, SW)], gsems[b]).wait()
                pltpu.make_async_copy(dst_hbm.at[s].at[j], didx_v.at[b],
                                      dsems[b]).wait()
                pltpu.sync_copy(rows_v.at[b], acc_sh.at[didx_v.at[b]], add=True)

        plsc.subcore_barrier()
        pltpu.sync_copy(acc_sh.at[pl.ds(s * rpt, rpt)],
                        out_hbm.at[c].at[pl.ds(s * rpt, rpt)])
        if tail:
            @pl.when(s == 0)
            def _out_tail():
                pltpu.sync_copy(acc_sh.at[pl.ds(NS * rpt, tail)],
                                out_hbm.at[c].at[pl.ds(NS * rpt, tail)])

    return agg_kernel


def _dinv_of(deg_blk):
    return lax.rsqrt(jnp.sum(deg_blk, axis=1, keepdims=True) + 1.0)


def _lin1_body(deg_ref, x_ref, w_ref, out_ref):
    dinv = _dinv_of(deg_ref[...])
    h = jnp.dot(x_ref[...], w_ref[...], preferred_element_type=jnp.float32)
    hn = h * dinv
    half = out_ref.shape[2]
    out_ref[0] = hn[:, :half]
    out_ref[1] = hn[:, half:]


def _lin2_body(deg_ref, agg_ref, b_ref, w_ref, out_ref):
    dinv = _dinv_of(deg_ref[...])
    agg = jnp.concatenate([agg_ref[0], agg_ref[1]], axis=1)
    x2 = jnp.maximum(agg * dinv + b_ref[...], 0.0)
    h = jnp.dot(x2, w_ref[...], preferred_element_type=jnp.float32)
    hn = h * dinv
    half = out_ref.shape[2]
    out_ref[0] = hn[:, :half]
    out_ref[1] = hn[:, half:]


def _lin3_body(deg_ref, agg_ref, b_ref, w_ref, bo_ref, out_ref):
    dinv = _dinv_of(deg_ref[...])
    agg = jnp.concatenate([agg_ref[0], agg_ref[1]], axis=1)
    x2 = jnp.maximum(agg * dinv + b_ref[...], 0.0)
    out_ref[...] = (
        jnp.dot(x2, w_ref[...], preferred_element_type=jnp.float32) + bo_ref[...]
    )


def kernel(x, edge_index, W1, b1, W2, b2, Wout, bout):
    n, d_in = x.shape
    d_h = W1.shape[1]
    d_out = Wout.shape[1]
    half = d_h // 2
    e = edge_index.shape[1]

    grp = NS * CW * NBUF
    e_pad = ((e + grp - 1) // grp) * grp
    cht = e_pad // (NS * CW)
    deg_pad = ((n + 1 + 511) // 512) * 512
    nrows = ((n + 1 + 15) // 16) * 16

    src = edge_index[0].astype(jnp.int32)
    dst = edge_index[1].astype(jnp.int32)
    padlen = e_pad - e
    pad_ids = jnp.arange(padlen, dtype=jnp.int32)
    src_p = jnp.concatenate([src, pad_ids % n])
    dst_p = jnp.concatenate([dst, n + pad_ids % (nrows - n)])
    src3 = src_p.reshape(NS, cht, CW)
    dst3 = dst_p.reshape(NS, cht, CW)

    deg_all = _build_deg(e_pad, deg_pad)(dst_p)
    deg_t = deg_all[:, :n].T

    r = 1000
    nblk = n // r
    deg_spec = pl.BlockSpec((r, NC * NS), lambda i: (i, 0))
    halves_spec = pl.BlockSpec((NC, r, half), lambda i: (0, i, 0))
    full_w = lambda shape: pl.BlockSpec(shape, lambda i: (0, 0))

    lin1 = pl.pallas_call(
        _lin1_body,
        grid=(nblk,),
        in_specs=[deg_spec, pl.BlockSpec((r, d_in), lambda i: (i, 0)),
                  full_w((d_in, d_h))],
        out_specs=halves_spec,
        out_shape=jax.ShapeDtypeStruct((NC, n, half), jnp.float32),
    )
    lin2 = pl.pallas_call(
        _lin2_body,
        grid=(nblk,),
        in_specs=[deg_spec, halves_spec, full_w((1, d_h)), full_w((d_h, d_h))],
        out_specs=halves_spec,
        out_shape=jax.ShapeDtypeStruct((NC, n, half), jnp.float32),
    )
    lin3 = pl.pallas_call(
        _lin3_body,
        grid=(nblk,),
        in_specs=[deg_spec, halves_spec, full_w((1, d_h)), full_w((d_h, d_out)),
                  full_w((1, d_out))],
        out_specs=pl.BlockSpec((r, d_out), lambda i: (i, 0)),
        out_shape=jax.ShapeDtypeStruct((n, d_out), jnp.float32),
    )
    agg = _build_agg(n, e_pad, half, nrows)

    hn1 = lin1(deg_t, x, W1)
    agg1 = agg(hn1, src3, dst3)
    hn2 = lin2(deg_t, agg1, b1.reshape(1, -1), W2)
    agg2 = agg(hn2, src3, dst3)
    out = lin3(deg_t, agg2, b2.reshape(1, -1), Wout, bout.reshape(1, -1))
    return out
